# baseline (speedup 1.0000x reference)
import functools

import jax
import jax.numpy as jnp
from jax import lax
from jax.experimental import pallas as pl
from jax.experimental.pallas import tpu as pltpu
from jax.experimental.pallas import tpu_sc as plsc


def _lane_bcast(vec, lane):
    idx = jnp.full((16, 1), lane, jnp.int32)
    dnums = lax.GatherDimensionNumbers(
        offset_dims=(), collapsed_slice_dims=(0,), start_index_map=(0,))
    return lax.gather(vec, idx, dnums, (1,),
                      mode=lax.GatherScatterMode.PROMISE_IN_BOUNDS)


_NC = 2
_NS = 16
_NW = _NC * _NS
_CHUNK = 64


def _make_spmm(n_nodes, d, n_chunks):
    assert n_chunks % 4 == 0
    mesh = plsc.VectorSubcoreMesh(core_axis_name="c", subcore_axis_name="s",
                                  num_cores=_NC, num_subcores=_NS)
    rows_per_tile = (n_nodes // _NS) // 8 * 8
    tail_base = rows_per_tile * _NS
    tail_rows = n_nodes - tail_base

    @functools.partial(
        pl.kernel,
        out_type=jax.ShapeDtypeStruct((_NC, n_nodes, d), jnp.float32),
        mesh=mesh,
        scratch_types=[
            pltpu.VMEM_SHARED((n_nodes, d), jnp.float32),
            pltpu.VMEM((4, 3, _CHUNK), jnp.int32),
            pltpu.VMEM((_CHUNK, d), jnp.float32),
            pltpu.VMEM((_CHUNK, d), jnp.float32),
            pltpu.VMEM((_CHUNK, d), jnp.float32),
            pltpu.VMEM((_CHUNK, d), jnp.float32),
            pltpu.SemaphoreType.DMA,
            pltpu.SemaphoreType.DMA,
            pltpu.SemaphoreType.DMA,
            pltpu.SemaphoreType.DMA,
            pltpu.SemaphoreType.DMA,
            pltpu.SemaphoreType.DMA,
            pltpu.SemaphoreType.DMA,
            pltpu.SemaphoreType.DMA,
        ],
        compiler_params=pltpu.CompilerParams(use_tc_tiling_on_sc=False,
                                             needs_layout_passes=False),
    )
    def spmm(h_hbm, edata_hbm, zeros_hbm, out_hbm,
             acc_sh, ebuf, g0, g1, s0, s1,
             esem0, esem1, esem2, esem3, gsem0, gsem1, ssem0, ssem1):
        gbufs, sbufs = (g0, g1), (s0, s1)
        esems = (esem0, esem1, esem2, esem3)
        gsems, ssems = (gsem0, gsem1), (ssem0, ssem1)
        c = lax.axis_index("c")
        s = lax.axis_index("s")
        wid = c * _NS + s

        pltpu.sync_copy(zeros_hbm.at[pl.ds(s * rows_per_tile, rows_per_tile)],
                        acc_sh.at[pl.ds(s * rows_per_tile, rows_per_tile)])
        if tail_rows:
            @pl.when(s == _NS - 1)
            def _():
                pltpu.sync_copy(zeros_hbm.at[pl.ds(tail_base, tail_rows)],
                                acc_sh.at[pl.ds(tail_base, tail_rows)])
        plsc.subcore_barrier()

        for b in range(2):
            pltpu.async_copy(edata_hbm.at[wid, b], ebuf.at[b], esems[b])
        for b in range(2):
            pltpu.make_async_copy(edata_hbm.at[wid, b], ebuf.at[b],
                                  esems[b]).wait()
            pltpu.async_copy(h_hbm.at[ebuf.at[b, 0]], gbufs[b], gsems[b])

        def quad_body(j4, carry):
            for b in range(4):
                chunk = j4 * 4 + b
                bb = b % 2
                gb, sb = gbufs[bb], sbufs[bb]
                pltpu.make_async_copy(h_hbm.at[ebuf.at[b, 0]], gb,
                                      gsems[bb]).wait()

                @pl.when(chunk >= 2)
                def _():
                    pltpu.make_async_copy(sb, acc_sh.at[ebuf.at[b, 1]],
                                          ssems[bb]).wait()

                @pl.when(chunk + 2 < n_chunks)
                def _():
                    pltpu.async_copy(edata_hbm.at[wid, chunk + 2],
                                     ebuf.at[(b + 2) % 4], esems[(b + 2) % 4])

                for g in range(_CHUNK // 16):
                    wv = plsc.bitcast(ebuf[b, 2, pl.ds(g * 16, 16)],
                                      jnp.float32)
                    for e16 in range(16):
                        e = g * 16 + e16
                        wb = _lane_bcast(wv, e16)
                        for f in range(d // 16):
                            sl = pl.ds(f * 16, 16)
                            sb[e, sl] = gb[e, sl] * wb

                @pl.when(chunk + 2 < n_chunks)
                def _():
                    pltpu.make_async_copy(edata_hbm.at[wid, 0],
                                          ebuf.at[(b + 2) % 4],
                                          esems[(b + 2) % 4]).wait()
                    pltpu.async_copy(h_hbm.at[ebuf.at[(b + 2) % 4, 0]], gb,
                                     gsems[bb])

                pltpu.async_copy(sb, acc_sh.at[ebuf.at[b, 1]], ssems[bb],
                                 add=True)
            return carry

        lax.fori_loop(0, n_chunks // 4, quad_body, 0)
        for b in range(2):
            pltpu.make_async_copy(sbufs[b], acc_sh.at[ebuf.at[b, 1]],
                                  ssems[b]).wait()

        plsc.subcore_barrier()
        pltpu.sync_copy(acc_sh.at[pl.ds(s * rows_per_tile, rows_per_tile)],
                        out_hbm.at[c, pl.ds(s * rows_per_tile, rows_per_tile)])
        if tail_rows:
            @pl.when(s == _NS - 1)
            def _():
                pltpu.sync_copy(acc_sh.at[pl.ds(tail_base, tail_rows)],
                                out_hbm.at[c, pl.ds(tail_base, tail_rows)])

    return spmm


def _dense_in(x, W_nbr, W_own, b, block_n=1000):
    n, d_in = x.shape
    d_out = W_nbr.shape[1]

    def body(x_ref, wn_ref, wo_ref, b_ref, h_ref, own_ref):
        xb = x_ref[...]
        h_ref[...] = jnp.dot(xb, wn_ref[...], preferred_element_type=jnp.float32)
        own_ref[...] = (
            jnp.dot(xb, wo_ref[...], preferred_element_type=jnp.float32)
            + b_ref[...]
        )

    return pl.pallas_call(
        body,
        grid=(n // block_n,),
        in_specs=[
            pl.BlockSpec((block_n, d_in), lambda i: (i, 0)),
            pl.BlockSpec((d_in, d_out), lambda i: (0, 0)),
            pl.BlockSpec((d_in, d_out), lambda i: (0, 0)),
            pl.BlockSpec((1, d_out), lambda i: (0, 0)),
        ],
        out_specs=[
            pl.BlockSpec((block_n, d_out), lambda i: (i, 0)),
            pl.BlockSpec((block_n, d_out), lambda i: (i, 0)),
        ],
        out_shape=[
            jax.ShapeDtypeStruct((n, d_out), jnp.float32),
            jax.ShapeDtypeStruct((n, d_out), jnp.float32),
        ],
    )(x, W_nbr, W_own, b.reshape(1, d_out))


def _dense_mid(parts, own0, W_nbr, W_own, b, block_n=1000):
    _, n, d_in = parts.shape
    d_out = W_nbr.shape[1]

    def body(p_ref, own_ref, wn_ref, wo_ref, b_ref, h1_ref, own1_ref):
        h = jnp.tanh(p_ref[0] + p_ref[1] + own_ref[...])
        h1_ref[...] = jnp.dot(h, wn_ref[...], preferred_element_type=jnp.float32)
        own1_ref[...] = (
            jnp.dot(h, wo_ref[...], preferred_element_type=jnp.float32)
            + b_ref[...]
        )

    return pl.pallas_call(
        body,
        grid=(n // block_n,),
        in_specs=[
            pl.BlockSpec((2, block_n, d_in), lambda i: (0, i, 0)),
            pl.BlockSpec((block_n, d_in), lambda i: (i, 0)),
            pl.BlockSpec((d_in, d_out), lambda i: (0, 0)),
            pl.BlockSpec((d_in, d_out), lambda i: (0, 0)),
            pl.BlockSpec((1, d_out), lambda i: (0, 0)),
        ],
        out_specs=[
            pl.BlockSpec((block_n, d_out), lambda i: (i, 0)),
            pl.BlockSpec((block_n, d_out), lambda i: (i, 0)),
        ],
        out_shape=[
            jax.ShapeDtypeStruct((n, d_out), jnp.float32),
            jax.ShapeDtypeStruct((n, d_out), jnp.float32),
        ],
    )(parts, own0, W_nbr, W_own, b.reshape(1, d_out))


def _dense_out(parts, own, block_n=1000):
    _, n, d = parts.shape

    def body(p_ref, own_ref, o_ref):
        o_ref[...] = p_ref[0] + p_ref[1] + own_ref[...]

    return pl.pallas_call(
        body,
        grid=(n // block_n,),
        in_specs=[
            pl.BlockSpec((2, block_n, d), lambda i: (0, i, 0)),
            pl.BlockSpec((block_n, d), lambda i: (i, 0)),
        ],
        out_specs=pl.BlockSpec((block_n, d), lambda i: (i, 0)),
        out_shape=jax.ShapeDtypeStruct((n, d), jnp.float32),
    )(parts, own)


def kernel(x, edge_index, edge_weight, W_own0, W_nbr0, b0, W_own1, W_nbr1, b1):
    n, d_in = x.shape
    e = edge_weight.shape[0]
    d_hid = W_nbr0.shape[1]
    d_out = W_nbr1.shape[1]

    n_chunks = -(-e // (_NW * _CHUNK))
    n_chunks = -(-n_chunks // 4) * 4
    per_worker = n_chunks * _CHUNK
    e_pad = per_worker * _NW

    dst = edge_index[0].astype(jnp.int32)
    src = edge_index[1].astype(jnp.int32)
    w_bits = lax.bitcast_convert_type(edge_weight.astype(jnp.float32),
                                      jnp.int32)
    pad = e_pad - e
    dst = jnp.concatenate([dst, jnp.zeros((pad,), jnp.int32)])
    src = jnp.concatenate([src, jnp.zeros((pad,), jnp.int32)])
    w_bits = jnp.concatenate([w_bits, jnp.zeros((pad,), jnp.int32)])
    edata = jnp.stack(
        [src.reshape(_NW, n_chunks, _CHUNK),
         dst.reshape(_NW, n_chunks, _CHUNK),
         w_bits.reshape(_NW, n_chunks, _CHUNK)], axis=2)

    zeros_hid = jnp.zeros((n, d_hid), jnp.float32)
    zeros_out = jnp.zeros((n, d_out), jnp.float32)

    spmm0 = _make_spmm(n, d_hid, n_chunks)
    spmm1 = _make_spmm(n, d_out, n_chunks)

    h0, own0 = _dense_in(x, W_nbr0, W_own0, b0)
    parts0 = spmm0(h0, edata, zeros_hid)
    h1, own1 = _dense_mid(parts0, own0, W_nbr1, W_own1, b1)
    parts1 = spmm1(h1, edata, zeros_out)
    return _dense_out(parts1, own1)


# speedup vs baseline: 1.8838x; 1.8838x over previous
import functools

import jax
import jax.numpy as jnp
from jax import lax
from jax.experimental import pallas as pl
from jax.experimental.pallas import tpu as pltpu
from jax.experimental.pallas import tpu_sc as plsc


def _lane_bcast(vec, lane):
    idx = jnp.full((16, 1), lane, jnp.int32)
    dnums = lax.GatherDimensionNumbers(
        offset_dims=(), collapsed_slice_dims=(0,), start_index_map=(0,))
    return lax.gather(vec, idx, dnums, (1,),
                      mode=lax.GatherScatterMode.PROMISE_IN_BOUNDS)


_NC = 2
_NS = 16
_CHUNK = 128
_IDXBITS = 14


def _make_spmm(n_nodes, d_half, n_chunks):
    assert n_chunks % 2 == 0
    mesh = plsc.VectorSubcoreMesh(core_axis_name="c", subcore_axis_name="s",
                                  num_cores=_NC, num_subcores=_NS)
    rows_per_tile = (n_nodes // _NS) // 8 * 8
    tail_base = rows_per_tile * _NS
    tail_rows = n_nodes - tail_base
    per_tile = n_chunks * _CHUNK

    @functools.partial(
        pl.kernel,
        out_type=jax.ShapeDtypeStruct((_NC, n_nodes, d_half), jnp.float32),
        mesh=mesh,
        scratch_types=[
            pltpu.VMEM_SHARED((n_nodes, d_half), jnp.float32),
            pltpu.VMEM((per_tile,), jnp.int32),
            pltpu.VMEM((per_tile,), jnp.float32),
            pltpu.VMEM((_CHUNK,), jnp.int32),
            pltpu.VMEM((_CHUNK,), jnp.int32),
            pltpu.VMEM((_CHUNK,), jnp.int32),
            pltpu.VMEM((_CHUNK,), jnp.int32),
            pltpu.VMEM((_CHUNK, d_half), jnp.float32),
            pltpu.VMEM((_CHUNK, d_half), jnp.float32),
            pltpu.VMEM((_CHUNK, d_half), jnp.float32),
            pltpu.VMEM((_CHUNK, d_half), jnp.float32),
            pltpu.SemaphoreType.DMA,
            pltpu.SemaphoreType.DMA,
            pltpu.SemaphoreType.DMA,
            pltpu.SemaphoreType.DMA,
        ],
        compiler_params=pltpu.CompilerParams(use_tc_tiling_on_sc=False,
                                             needs_layout_passes=False),
    )
    def spmm(h_hbm, ep_hbm, w_hbm, zeros_hbm, out_hbm,
             acc_sh, ep_v, w_v, si0, si1, di0, di1, g0, g1, s0, s1,
             gsem0, gsem1, ssem0, ssem1):
        gbufs, sbufs = (g0, g1), (s0, s1)
        sidx, didx = (si0, si1), (di0, di1)
        gsems, ssems = (gsem0, gsem1), (ssem0, ssem1)
        c = lax.axis_index("c")
        s = lax.axis_index("s")
        row_ofs = c * n_nodes

        pltpu.sync_copy(zeros_hbm.at[pl.ds(s * rows_per_tile, rows_per_tile)],
                        acc_sh.at[pl.ds(s * rows_per_tile, rows_per_tile)])
        if tail_rows:
            @pl.when(s == _NS - 1)
            def _():
                pltpu.sync_copy(zeros_hbm.at[pl.ds(tail_base, tail_rows)],
                                acc_sh.at[pl.ds(tail_base, tail_rows)])

        pltpu.sync_copy(ep_hbm.at[s], ep_v)
        pltpu.sync_copy(w_hbm.at[s], w_v)
        plsc.subcore_barrier()

        def build_sidx(chunk, buf):
            for g16 in range(_CHUNK // 16):
                sl = pl.ds(g16 * 16, 16)
                v = ep_v[pl.ds(chunk * _CHUNK + g16 * 16, 16)]
                buf[sl] = (v & ((1 << _IDXBITS) - 1)) + row_ofs

        def build_didx(chunk, buf):
            for g16 in range(_CHUNK // 16):
                sl = pl.ds(g16 * 16, 16)
                v = ep_v[pl.ds(chunk * _CHUNK + g16 * 16, 16)]
                buf[sl] = lax.shift_right_logical(v, _IDXBITS)

        for b in range(2):
            build_sidx(jnp.int32(b), sidx[b])
            pltpu.async_copy(h_hbm.at[sidx[b]], gbufs[b], gsems[b])

        def pair_body(j2, carry):
            for b in range(2):
                chunk = j2 * 2 + b
                gb, sb = gbufs[b], sbufs[b]
                pltpu.make_async_copy(h_hbm.at[sidx[b]], gb, gsems[b]).wait()

                @pl.when(chunk >= 2)
                def _():
                    pltpu.make_async_copy(sb, acc_sh.at[didx[b]],
                                          ssems[b]).wait()

                for g in range(_CHUNK // 16):
                    wv = w_v[pl.ds(chunk * _CHUNK + g * 16, 16)]
                    for e16 in range(16):
                        e = g * 16 + e16
                        wb = _lane_bcast(wv, e16)
                        for f in range(d_half // 16):
                            sl = pl.ds(f * 16, 16)
                            sb[e, sl] = gb[e, sl] * wb

                @pl.when(chunk + 2 < n_chunks)
                def _():
                    build_sidx(chunk + 2, sidx[b])
                    pltpu.async_copy(h_hbm.at[sidx[b]], gb, gsems[b])

                build_didx(chunk, didx[b])
                pltpu.async_copy(sb, acc_sh.at[didx[b]], ssems[b], add=True)
            return carry

        lax.fori_loop(0, n_chunks // 2, pair_body, 0)
        for b in range(2):
            pltpu.make_async_copy(sbufs[b], acc_sh.at[didx[b]],
                                  ssems[b]).wait()

        plsc.subcore_barrier()
        pltpu.sync_copy(acc_sh.at[pl.ds(s * rows_per_tile, rows_per_tile)],
                        out_hbm.at[c, pl.ds(s * rows_per_tile, rows_per_tile)])
        if tail_rows:
            @pl.when(s == _NS - 1)
            def _():
                pltpu.sync_copy(acc_sh.at[pl.ds(tail_base, tail_rows)],
                                out_hbm.at[c, pl.ds(tail_base, tail_rows)])

    return spmm


def _nbr_matmul(x, W_nbr, block_n=1000):
    n, d_in = x.shape
    d_out = W_nbr.shape[1]
    dh = d_out // 2

    W2 = jnp.stack([W_nbr[:, :dh], W_nbr[:, dh:]])

    def body(x_ref, wn_ref, h_ref):
        h_ref[...] = jnp.dot(x_ref[...], wn_ref[0],
                             preferred_element_type=jnp.float32)

    return pl.pallas_call(
        body,
        grid=(2, n // block_n),
        in_specs=[
            pl.BlockSpec((block_n, d_in), lambda j, i: (i, 0)),
            pl.BlockSpec((1, d_in, dh), lambda j, i: (j, 0, 0)),
        ],
        out_specs=pl.BlockSpec((block_n, dh),
                               lambda j, i: (j * (n // block_n) + i, 0)),
        out_shape=jax.ShapeDtypeStruct((2 * n, dh), jnp.float32),
    )(x, W2)


def _own_matmul(x, W_own, b, block_n=1000):
    n, d_in = x.shape
    d_out = W_own.shape[1]

    def body(x_ref, wo_ref, b_ref, o_ref):
        o_ref[...] = (
            jnp.dot(x_ref[...], wo_ref[...], preferred_element_type=jnp.float32)
            + b_ref[...]
        )

    return pl.pallas_call(
        body,
        grid=(n // block_n,),
        in_specs=[
            pl.BlockSpec((block_n, d_in), lambda i: (i, 0)),
            pl.BlockSpec((d_in, d_out), lambda i: (0, 0)),
            pl.BlockSpec((1, d_out), lambda i: (0, 0)),
        ],
        out_specs=pl.BlockSpec((block_n, d_out), lambda i: (i, 0)),
        out_shape=jax.ShapeDtypeStruct((n, d_out), jnp.float32),
    )(x, W_own, b.reshape(1, d_out))


def _mid_tanh(parts, own, block_n=1000):
    _, n, dh = parts.shape
    d = 2 * dh

    def body(p_ref, own_ref, h_ref):
        agg = jnp.concatenate([p_ref[0], p_ref[1]], axis=1)
        h_ref[...] = jnp.tanh(agg + own_ref[...])

    return pl.pallas_call(
        body,
        grid=(n // block_n,),
        in_specs=[
            pl.BlockSpec((2, block_n, dh), lambda i: (0, i, 0)),
            pl.BlockSpec((block_n, d), lambda i: (i, 0)),
        ],
        out_specs=pl.BlockSpec((block_n, d), lambda i: (i, 0)),
        out_shape=jax.ShapeDtypeStruct((n, d), jnp.float32),
    )(parts, own)


def _final_concat_add(parts, own, block_n=1000):
    _, n, dh = parts.shape
    d = 2 * dh

    def body(p_ref, own_ref, o_ref):
        agg = jnp.concatenate([p_ref[0], p_ref[1]], axis=1)
        o_ref[...] = agg + own_ref[...]

    return pl.pallas_call(
        body,
        grid=(n // block_n,),
        in_specs=[
            pl.BlockSpec((2, block_n, dh), lambda i: (0, i, 0)),
            pl.BlockSpec((block_n, d), lambda i: (i, 0)),
        ],
        out_specs=pl.BlockSpec((block_n, d), lambda i: (i, 0)),
        out_shape=jax.ShapeDtypeStruct((n, d), jnp.float32),
    )(parts, own)


def kernel(x, edge_index, edge_weight, W_own0, W_nbr0, b0, W_own1, W_nbr1, b1):
    n, d_in = x.shape
    e = edge_weight.shape[0]
    d_hid = W_nbr0.shape[1]
    d_out = W_nbr1.shape[1]

    n_chunks = -(-e // (_NS * _CHUNK))
    n_chunks += n_chunks % 2
    per_tile = n_chunks * _CHUNK
    e_pad = per_tile * _NS

    dst = edge_index[0].astype(jnp.int32)
    src = edge_index[1].astype(jnp.int32)
    w = edge_weight.astype(jnp.float32)
    pad = e_pad - e
    ep = src | (dst << _IDXBITS)
    ep = jnp.concatenate([ep, jnp.zeros((pad,), jnp.int32)])
    w = jnp.concatenate([w, jnp.zeros((pad,), jnp.float32)])
    ep = ep.reshape(_NS, per_tile)
    w = w.reshape(_NS, per_tile)

    zeros_hid = jnp.zeros((n, d_hid // 2), jnp.float32)
    zeros_out = jnp.zeros((n, d_out // 2), jnp.float32)

    spmm0 = _make_spmm(n, d_hid // 2, n_chunks)
    spmm1 = _make_spmm(n, d_out // 2, n_chunks)

    h0 = _nbr_matmul(x, W_nbr0)
    parts0 = spmm0(h0, ep, w, zeros_hid)
    own0 = _own_matmul(x, W_own0, b0)
    h = _mid_tanh(parts0, own0)
    h1 = _nbr_matmul(h, W_nbr1)
    parts1 = spmm1(h1, ep, w, zeros_out)
    own1 = _own_matmul(h, W_own1, b1)
    return _final_concat_add(parts1, own1)


# speedup vs baseline: 2.0626x; 1.0949x over previous
import functools

import jax
import jax.numpy as jnp
import numpy as np
from jax import lax
from jax.experimental import pallas as pl
from jax.experimental.pallas import tpu as pltpu
from jax.experimental.pallas import tpu_sc as plsc


def _lane_bcast(vec, lane):
    idx = jnp.full((16, 1), lane, jnp.int32)
    dnums = lax.GatherDimensionNumbers(
        offset_dims=(), collapsed_slice_dims=(0,), start_index_map=(0,))
    return lax.gather(vec, idx, dnums, (1,),
                      mode=lax.GatherScatterMode.PROMISE_IN_BOUNDS)


_NC = 2
_NS = 16
_CHUNK = 128
_IDXBITS = 14


def _make_spmm(n_nodes, d_half, n_chunks):
    assert n_chunks % 2 == 0
    mesh = plsc.VectorSubcoreMesh(core_axis_name="c", subcore_axis_name="s",
                                  num_cores=_NC, num_subcores=_NS)
    rows_per_tile = (n_nodes // _NS) // 8 * 8
    tail_base = rows_per_tile * _NS
    tail_rows = n_nodes - tail_base
    per_tile = n_chunks * _CHUNK

    @functools.partial(
        pl.kernel,
        out_type=jax.ShapeDtypeStruct((_NC, n_nodes, d_half), jnp.float32),
        mesh=mesh,
        scratch_types=[
            pltpu.VMEM_SHARED((n_nodes, d_half), jnp.float32),
            pltpu.VMEM((per_tile,), jnp.int32),
            pltpu.VMEM((per_tile,), jnp.float32),
            pltpu.VMEM((_CHUNK,), jnp.int32),
            pltpu.VMEM((_CHUNK,), jnp.int32),
            pltpu.VMEM((_CHUNK,), jnp.int32),
            pltpu.VMEM((_CHUNK,), jnp.int32),
            pltpu.VMEM((_CHUNK, d_half // 2), jnp.int32),
            pltpu.VMEM((_CHUNK, d_half // 2), jnp.int32),
            pltpu.VMEM((_CHUNK, d_half), jnp.float32),
            pltpu.VMEM((_CHUNK, d_half), jnp.float32),
            pltpu.SemaphoreType.DMA,
            pltpu.SemaphoreType.DMA,
            pltpu.SemaphoreType.DMA,
            pltpu.SemaphoreType.DMA,
        ],
        compiler_params=pltpu.CompilerParams(use_tc_tiling_on_sc=False,
                                             needs_layout_passes=False),
    )
    def spmm(h_hbm, ep_hbm, w_hbm, zeros_hbm, out_hbm,
             acc_sh, ep_v, w_v, si0, si1, di0, di1, g0, g1, s0, s1,
             gsem0, gsem1, ssem0, ssem1):
        gbufs, sbufs = (g0, g1), (s0, s1)
        sidx, didx = (si0, si1), (di0, di1)
        gsems, ssems = (gsem0, gsem1), (ssem0, ssem1)
        c = lax.axis_index("c")
        s = lax.axis_index("s")
        row_ofs = c * n_nodes

        pltpu.sync_copy(zeros_hbm.at[pl.ds(s * rows_per_tile, rows_per_tile)],
                        acc_sh.at[pl.ds(s * rows_per_tile, rows_per_tile)])
        if tail_rows:
            @pl.when(s == _NS - 1)
            def _():
                pltpu.sync_copy(zeros_hbm.at[pl.ds(tail_base, tail_rows)],
                                acc_sh.at[pl.ds(tail_base, tail_rows)])

        pltpu.sync_copy(ep_hbm.at[s], ep_v)
        pltpu.sync_copy(w_hbm.at[s], w_v)
        plsc.subcore_barrier()

        def build_sidx(chunk, buf):
            for g16 in range(_CHUNK // 16):
                sl = pl.ds(g16 * 16, 16)
                v = ep_v[pl.ds(chunk * _CHUNK + g16 * 16, 16)]
                buf[sl] = (v & ((1 << _IDXBITS) - 1)) + row_ofs

        def build_didx(chunk, buf):
            for g16 in range(_CHUNK // 16):
                sl = pl.ds(g16 * 16, 16)
                v = ep_v[pl.ds(chunk * _CHUNK + g16 * 16, 16)]
                buf[sl] = lax.shift_right_logical(v, _IDXBITS)

        for b in range(2):
            build_sidx(jnp.int32(b), sidx[b])
            pltpu.async_copy(h_hbm.at[sidx[b]], gbufs[b], gsems[b])

        def pair_body(j2, carry):
            for b in range(2):
                chunk = j2 * 2 + b
                gb, sb = gbufs[b], sbufs[b]
                pltpu.make_async_copy(h_hbm.at[sidx[b]], gb, gsems[b]).wait()

                @pl.when(chunk >= 2)
                def _():
                    pltpu.make_async_copy(sb, acc_sh.at[didx[b]],
                                          ssems[b]).wait()

                for g in range(_CHUNK // 16):
                    wv = w_v[pl.ds(chunk * _CHUNK + g * 16, 16)]
                    for e16 in range(16):
                        e = g * 16 + e16
                        wb = _lane_bcast(wv, e16)
                        for f in range(d_half // 32):
                            v = gb[e, pl.ds(f * 16, 16)]
                            lo = plsc.bitcast(lax.shift_left(v, 16),
                                              jnp.float32)
                            hi = plsc.bitcast(v & jnp.int32(-65536),
                                              jnp.float32)
                            sb[e, pl.ds(f * 32, 16)] = lo * wb
                            sb[e, pl.ds(f * 32 + 16, 16)] = hi * wb

                @pl.when(chunk + 2 < n_chunks)
                def _():
                    build_sidx(chunk + 2, sidx[b])
                    pltpu.async_copy(h_hbm.at[sidx[b]], gb, gsems[b])

                build_didx(chunk, didx[b])
                pltpu.async_copy(sb, acc_sh.at[didx[b]], ssems[b], add=True)
            return carry

        lax.fori_loop(0, n_chunks // 2, pair_body, 0)
        for b in range(2):
            pltpu.make_async_copy(sbufs[b], acc_sh.at[didx[b]],
                                  ssems[b]).wait()

        plsc.subcore_barrier()
        pltpu.sync_copy(acc_sh.at[pl.ds(s * rows_per_tile, rows_per_tile)],
                        out_hbm.at[c, pl.ds(s * rows_per_tile, rows_per_tile)])
        if tail_rows:
            @pl.when(s == _NS - 1)
            def _():
                pltpu.sync_copy(acc_sh.at[pl.ds(tail_base, tail_rows)],
                                out_hbm.at[c, pl.ds(tail_base, tail_rows)])

    return spmm


def _pack_colorder(dh):
    co = np.empty(dh, np.int64)
    for f in range(dh // 32):
        for l in range(16):
            co[32 * f + 2 * l] = 32 * f + l
            co[32 * f + 2 * l + 1] = 32 * f + 16 + l
    return co


def _nbr_matmul(x, W_nbr, block_n=1000):
    n, d_in = x.shape
    d_out = W_nbr.shape[1]
    dh = d_out // 2

    co = _pack_colorder(dh)
    W2 = jnp.stack([W_nbr[:, :dh][:, co], W_nbr[:, dh:][:, co]])

    def body(x_ref, wn_ref, h_ref):
        h_ref[...] = jnp.dot(
            x_ref[...], wn_ref[0], preferred_element_type=jnp.float32
        ).astype(jnp.bfloat16)

    hbf = pl.pallas_call(
        body,
        grid=(2, n // block_n),
        in_specs=[
            pl.BlockSpec((block_n, d_in), lambda j, i: (i, 0)),
            pl.BlockSpec((1, d_in, dh), lambda j, i: (j, 0, 0)),
        ],
        out_specs=pl.BlockSpec((block_n, dh),
                               lambda j, i: (j * (n // block_n) + i, 0)),
        out_shape=jax.ShapeDtypeStruct((2 * n, dh), jnp.bfloat16),
    )(x, W2)
    return lax.bitcast_convert_type(hbf.reshape(2 * n, dh // 2, 2), jnp.int32)


def _own_matmul(x, W_own, b, block_n=1000):
    n, d_in = x.shape
    d_out = W_own.shape[1]

    def body(x_ref, wo_ref, b_ref, o_ref):
        o_ref[...] = (
            jnp.dot(x_ref[...], wo_ref[...], preferred_element_type=jnp.float32)
            + b_ref[...]
        )

    return pl.pallas_call(
        body,
        grid=(n // block_n,),
        in_specs=[
            pl.BlockSpec((block_n, d_in), lambda i: (i, 0)),
            pl.BlockSpec((d_in, d_out), lambda i: (0, 0)),
            pl.BlockSpec((1, d_out), lambda i: (0, 0)),
        ],
        out_specs=pl.BlockSpec((block_n, d_out), lambda i: (i, 0)),
        out_shape=jax.ShapeDtypeStruct((n, d_out), jnp.float32),
    )(x, W_own, b.reshape(1, d_out))


def _mid_tanh(parts, own, block_n=1000):
    _, n, dh = parts.shape
    d = 2 * dh

    def body(p_ref, own_ref, h_ref):
        agg = jnp.concatenate([p_ref[0], p_ref[1]], axis=1)
        h_ref[...] = jnp.tanh(agg + own_ref[...])

    return pl.pallas_call(
        body,
        grid=(n // block_n,),
        in_specs=[
            pl.BlockSpec((2, block_n, dh), lambda i: (0, i, 0)),
            pl.BlockSpec((block_n, d), lambda i: (i, 0)),
        ],
        out_specs=pl.BlockSpec((block_n, d), lambda i: (i, 0)),
        out_shape=jax.ShapeDtypeStruct((n, d), jnp.float32),
    )(parts, own)


def _final_concat_add(parts, own, block_n=1000):
    _, n, dh = parts.shape
    d = 2 * dh

    def body(p_ref, own_ref, o_ref):
        agg = jnp.concatenate([p_ref[0], p_ref[1]], axis=1)
        o_ref[...] = agg + own_ref[...]

    return pl.pallas_call(
        body,
        grid=(n // block_n,),
        in_specs=[
            pl.BlockSpec((2, block_n, dh), lambda i: (0, i, 0)),
            pl.BlockSpec((block_n, d), lambda i: (i, 0)),
        ],
        out_specs=pl.BlockSpec((block_n, d), lambda i: (i, 0)),
        out_shape=jax.ShapeDtypeStruct((n, d), jnp.float32),
    )(parts, own)


def kernel(x, edge_index, edge_weight, W_own0, W_nbr0, b0, W_own1, W_nbr1, b1):
    n, d_in = x.shape
    e = edge_weight.shape[0]
    d_hid = W_nbr0.shape[1]
    d_out = W_nbr1.shape[1]

    n_chunks = -(-e // (_NS * _CHUNK))
    n_chunks += n_chunks % 2
    per_tile = n_chunks * _CHUNK
    e_pad = per_tile * _NS

    dst = edge_index[0].astype(jnp.int32)
    src = edge_index[1].astype(jnp.int32)
    w = edge_weight.astype(jnp.float32)
    pad = e_pad - e
    ep = src | (dst << _IDXBITS)
    ep = jnp.concatenate([ep, jnp.zeros((pad,), jnp.int32)])
    w = jnp.concatenate([w, jnp.zeros((pad,), jnp.float32)])
    ep = ep.reshape(_NS, per_tile)
    w = w.reshape(_NS, per_tile)

    zeros_hid = jnp.zeros((n, d_hid // 2), jnp.float32)
    zeros_out = jnp.zeros((n, d_out // 2), jnp.float32)

    spmm0 = _make_spmm(n, d_hid // 2, n_chunks)
    spmm1 = _make_spmm(n, d_out // 2, n_chunks)

    h0 = _nbr_matmul(x, W_nbr0)
    parts0 = spmm0(h0, ep, w, zeros_hid)
    own0 = _own_matmul(x, W_own0, b0)
    h = _mid_tanh(parts0, own0)
    h1 = _nbr_matmul(h, W_nbr1)
    parts1 = spmm1(h1, ep, w, zeros_out)
    own1 = _own_matmul(h, W_own1, b1)
    return _final_concat_add(parts1, own1)


# speedup vs baseline: 2.1671x; 1.0507x over previous
import functools

import jax
import jax.numpy as jnp
import numpy as np
from jax import lax
from jax.experimental import pallas as pl
from jax.experimental.pallas import tpu as pltpu
from jax.experimental.pallas import tpu_sc as plsc


def _lane_bcast(vec, lane):
    idx = jnp.full((16, 1), lane, jnp.int32)
    dnums = lax.GatherDimensionNumbers(
        offset_dims=(), collapsed_slice_dims=(0,), start_index_map=(0,))
    return lax.gather(vec, idx, dnums, (1,),
                      mode=lax.GatherScatterMode.PROMISE_IN_BOUNDS)


_NC = 2
_NS = 16
_CHUNK = 128
_IDXBITS = 14


def _make_spmm(n_nodes, d_half, n_chunks):
    assert n_chunks % 2 == 0
    mesh = plsc.VectorSubcoreMesh(core_axis_name="c", subcore_axis_name="s",
                                  num_cores=_NC, num_subcores=_NS)
    rows_per_tile = (n_nodes // _NS) // 8 * 8
    tail_base = rows_per_tile * _NS
    tail_rows = n_nodes - tail_base
    per_tile = n_chunks * _CHUNK

    @functools.partial(
        pl.kernel,
        out_type=jax.ShapeDtypeStruct((_NC, n_nodes, d_half), jnp.float32),
        mesh=mesh,
        scratch_types=[
            pltpu.VMEM_SHARED((n_nodes, d_half), jnp.float32),
            pltpu.VMEM((per_tile,), jnp.int32),
            pltpu.VMEM((per_tile,), jnp.float32),
            pltpu.VMEM((_CHUNK,), jnp.int32),
            pltpu.VMEM((_CHUNK,), jnp.int32),
            pltpu.VMEM((_CHUNK,), jnp.int32),
            pltpu.VMEM((_CHUNK,), jnp.int32),
            pltpu.VMEM((_CHUNK, d_half), jnp.bfloat16),
            pltpu.VMEM((_CHUNK, d_half), jnp.bfloat16),
            pltpu.VMEM((_CHUNK, d_half), jnp.float32),
            pltpu.VMEM((_CHUNK, d_half), jnp.float32),
            pltpu.SemaphoreType.DMA,
            pltpu.SemaphoreType.DMA,
            pltpu.SemaphoreType.DMA,
            pltpu.SemaphoreType.DMA,
        ],
        compiler_params=pltpu.CompilerParams(use_tc_tiling_on_sc=False,
                                             needs_layout_passes=False),
    )
    def spmm(h_hbm, ep_hbm, w_hbm, zeros_hbm, out_hbm,
             acc_sh, ep_v, w_v, si0, si1, di0, di1, g0, g1, s0, s1,
             gsem0, gsem1, ssem0, ssem1):
        gbufs, sbufs = (g0, g1), (s0, s1)
        sidx, didx = (si0, si1), (di0, di1)
        gsems, ssems = (gsem0, gsem1), (ssem0, ssem1)
        c = lax.axis_index("c")
        s = lax.axis_index("s")
        row_ofs = c * n_nodes

        pltpu.sync_copy(zeros_hbm.at[pl.ds(s * rows_per_tile, rows_per_tile)],
                        acc_sh.at[pl.ds(s * rows_per_tile, rows_per_tile)])
        if tail_rows:
            @pl.when(s == _NS - 1)
            def _():
                pltpu.sync_copy(zeros_hbm.at[pl.ds(tail_base, tail_rows)],
                                acc_sh.at[pl.ds(tail_base, tail_rows)])

        pltpu.sync_copy(ep_hbm.at[pl.ds(s * per_tile, per_tile)], ep_v)
        pltpu.sync_copy(w_hbm.at[pl.ds(s * per_tile, per_tile)], w_v)
        plsc.subcore_barrier()

        def build_sidx(chunk, buf):
            for g16 in range(_CHUNK // 16):
                sl = pl.ds(g16 * 16, 16)
                v = ep_v[pl.ds(chunk * _CHUNK + g16 * 16, 16)]
                buf[sl] = (v & ((1 << _IDXBITS) - 1)) + row_ofs

        def build_didx(chunk, buf):
            for g16 in range(_CHUNK // 16):
                sl = pl.ds(g16 * 16, 16)
                v = ep_v[pl.ds(chunk * _CHUNK + g16 * 16, 16)]
                buf[sl] = lax.shift_right_logical(v, _IDXBITS)

        for b in range(2):
            build_sidx(jnp.int32(b), sidx[b])
            pltpu.async_copy(h_hbm.at[sidx[b]], gbufs[b], gsems[b])

        def pair_body(j2, carry):
            for b in range(2):
                chunk = j2 * 2 + b
                gb, sb = gbufs[b], sbufs[b]
                pltpu.make_async_copy(h_hbm.at[sidx[b]], gb, gsems[b]).wait()

                @pl.when(chunk >= 2)
                def _():
                    pltpu.make_async_copy(sb, acc_sh.at[didx[b]],
                                          ssems[b]).wait()

                for g in range(_CHUNK // 16):
                    wv = w_v[pl.ds(chunk * _CHUNK + g * 16, 16)]
                    for e16 in range(16):
                        e = g * 16 + e16
                        wb = _lane_bcast(wv, e16)
                        for f in range(d_half // 32):
                            v = plsc.bitcast(gb[e, pl.ds(f * 32, 32)],
                                             jnp.int32)
                            lo = plsc.bitcast(lax.shift_left(v, 16),
                                              jnp.float32)
                            hi = plsc.bitcast(v & jnp.int32(-65536),
                                              jnp.float32)
                            sb[e, pl.ds(f * 32, 16)] = lo * wb
                            sb[e, pl.ds(f * 32 + 16, 16)] = hi * wb

                @pl.when(chunk + 2 < n_chunks)
                def _():
                    build_sidx(chunk + 2, sidx[b])
                    pltpu.async_copy(h_hbm.at[sidx[b]], gb, gsems[b])

                build_didx(chunk, didx[b])
                pltpu.async_copy(sb, acc_sh.at[didx[b]], ssems[b], add=True)
            return carry

        lax.fori_loop(0, n_chunks // 2, pair_body, 0)
        for b in range(2):
            pltpu.make_async_copy(sbufs[b], acc_sh.at[didx[b]],
                                  ssems[b]).wait()

        plsc.subcore_barrier()
        pltpu.sync_copy(acc_sh.at[pl.ds(s * rows_per_tile, rows_per_tile)],
                        out_hbm.at[c, pl.ds(s * rows_per_tile, rows_per_tile)])
        if tail_rows:
            @pl.when(s == _NS - 1)
            def _():
                pltpu.sync_copy(acc_sh.at[pl.ds(tail_base, tail_rows)],
                                out_hbm.at[c, pl.ds(tail_base, tail_rows)])

    return spmm


def _pack_colorder(dh):
    co = np.empty(dh, np.int64)
    for f in range(dh // 32):
        for l in range(16):
            co[32 * f + 2 * l] = 32 * f + l
            co[32 * f + 2 * l + 1] = 32 * f + 16 + l
    return co


def _nbr_matmul(x, W_nbr, block_n=1000):
    n, d_in = x.shape
    d_out = W_nbr.shape[1]
    dh = d_out // 2

    co = _pack_colorder(dh)
    W2 = jnp.stack([W_nbr[:, :dh][:, co], W_nbr[:, dh:][:, co]])

    def body(x_ref, wn_ref, h_ref):
        h_ref[...] = jnp.dot(
            x_ref[...], wn_ref[0], preferred_element_type=jnp.float32
        ).astype(jnp.bfloat16)

    return pl.pallas_call(
        body,
        grid=(2, n // block_n),
        in_specs=[
            pl.BlockSpec((block_n, d_in), lambda j, i: (i, 0)),
            pl.BlockSpec((1, d_in, dh), lambda j, i: (j, 0, 0)),
        ],
        out_specs=pl.BlockSpec((block_n, dh),
                               lambda j, i: (j * (n // block_n) + i, 0)),
        out_shape=jax.ShapeDtypeStruct((2 * n, dh), jnp.bfloat16),
    )(x, W2)


def _edge_pack(src, dst, n_rows=None):
    (e_pad,) = src.shape
    rows = e_pad // 128
    block_r = rows
    for cand in range(min(rows, 512), 0, -1):
        if rows % cand == 0 and cand % 8 == 0:
            block_r = cand
            break

    def body(s_ref, d_ref, o_ref):
        o_ref[...] = s_ref[...] | (d_ref[...] << _IDXBITS)

    packed = pl.pallas_call(
        body,
        grid=(rows // block_r,),
        in_specs=[
            pl.BlockSpec((block_r, 128), lambda i: (i, 0)),
            pl.BlockSpec((block_r, 128), lambda i: (i, 0)),
        ],
        out_specs=pl.BlockSpec((block_r, 128), lambda i: (i, 0)),
        out_shape=jax.ShapeDtypeStruct((rows, 128), jnp.int32),
    )(src.reshape(rows, 128), dst.reshape(rows, 128))
    return packed.reshape(e_pad)


def _own_matmul(x, W_own, b, block_n=1000):
    n, d_in = x.shape
    d_out = W_own.shape[1]

    def body(x_ref, wo_ref, b_ref, o_ref):
        o_ref[...] = (
            jnp.dot(x_ref[...], wo_ref[...], preferred_element_type=jnp.float32)
            + b_ref[...]
        )

    return pl.pallas_call(
        body,
        grid=(n // block_n,),
        in_specs=[
            pl.BlockSpec((block_n, d_in), lambda i: (i, 0)),
            pl.BlockSpec((d_in, d_out), lambda i: (0, 0)),
            pl.BlockSpec((1, d_out), lambda i: (0, 0)),
        ],
        out_specs=pl.BlockSpec((block_n, d_out), lambda i: (i, 0)),
        out_shape=jax.ShapeDtypeStruct((n, d_out), jnp.float32),
    )(x, W_own, b.reshape(1, d_out))


def _mid_tanh(parts, own, block_n=1000):
    _, n, dh = parts.shape
    d = 2 * dh

    def body(p_ref, own_ref, h_ref):
        agg = jnp.concatenate([p_ref[0], p_ref[1]], axis=1)
        h_ref[...] = jnp.tanh(agg + own_ref[...])

    return pl.pallas_call(
        body,
        grid=(n // block_n,),
        in_specs=[
            pl.BlockSpec((2, block_n, dh), lambda i: (0, i, 0)),
            pl.BlockSpec((block_n, d), lambda i: (i, 0)),
        ],
        out_specs=pl.BlockSpec((block_n, d), lambda i: (i, 0)),
        out_shape=jax.ShapeDtypeStruct((n, d), jnp.float32),
    )(parts, own)


def _final_concat_add(parts, own, block_n=1000):
    _, n, dh = parts.shape
    d = 2 * dh

    def body(p_ref, own_ref, o_ref):
        agg = jnp.concatenate([p_ref[0], p_ref[1]], axis=1)
        o_ref[...] = agg + own_ref[...]

    return pl.pallas_call(
        body,
        grid=(n // block_n,),
        in_specs=[
            pl.BlockSpec((2, block_n, dh), lambda i: (0, i, 0)),
            pl.BlockSpec((block_n, d), lambda i: (i, 0)),
        ],
        out_specs=pl.BlockSpec((block_n, d), lambda i: (i, 0)),
        out_shape=jax.ShapeDtypeStruct((n, d), jnp.float32),
    )(parts, own)


def kernel(x, edge_index, edge_weight, W_own0, W_nbr0, b0, W_own1, W_nbr1, b1):
    n, d_in = x.shape
    e = edge_weight.shape[0]
    d_hid = W_nbr0.shape[1]
    d_out = W_nbr1.shape[1]

    n_chunks = -(-e // (_NS * _CHUNK))
    n_chunks += n_chunks % 2
    per_tile = n_chunks * _CHUNK
    e_pad = per_tile * _NS

    pad = e_pad - e
    zpad = jnp.zeros((pad,), jnp.int32)
    dst = jnp.concatenate([edge_index[0].astype(jnp.int32), zpad])
    src = jnp.concatenate([edge_index[1].astype(jnp.int32), zpad])
    w = jnp.concatenate([edge_weight.astype(jnp.float32),
                         jnp.zeros((pad,), jnp.float32)])
    ep = _edge_pack(src, dst)

    zeros_hid = jnp.zeros((n, d_hid // 2), jnp.float32)
    zeros_out = jnp.zeros((n, d_out // 2), jnp.float32)

    spmm0 = _make_spmm(n, d_hid // 2, n_chunks)
    spmm1 = _make_spmm(n, d_out // 2, n_chunks)

    h0 = _nbr_matmul(x, W_nbr0)
    parts0 = spmm0(h0, ep, w, zeros_hid)
    own0 = _own_matmul(x, W_own0, b0)
    h = _mid_tanh(parts0, own0)
    h1 = _nbr_matmul(h, W_nbr1)
    parts1 = spmm1(h1, ep, w, zeros_out)
    own1 = _own_matmul(h, W_own1, b1)
    return _final_concat_add(parts1, own1)


# speedup vs baseline: 2.4778x; 1.1434x over previous
import functools

import jax
import jax.numpy as jnp
import numpy as np
from jax import lax
from jax.experimental import pallas as pl
from jax.experimental.pallas import tpu as pltpu
from jax.experimental.pallas import tpu_sc as plsc


def _lane_bcast(vec, lane):
    idx = jnp.full((16, 1), lane, jnp.int32)
    dnums = lax.GatherDimensionNumbers(
        offset_dims=(), collapsed_slice_dims=(0,), start_index_map=(0,))
    return lax.gather(vec, idx, dnums, (1,),
                      mode=lax.GatherScatterMode.PROMISE_IN_BOUNDS)


_NC = 2
_NS = 16
_CHUNK = 128
_IDXBITS = 14


def _make_spmm(n_nodes, d_half, n_chunks):
    assert n_chunks % 2 == 0
    mesh = plsc.VectorSubcoreMesh(core_axis_name="c", subcore_axis_name="s",
                                  num_cores=_NC, num_subcores=_NS)
    rows_per_tile = (n_nodes // _NS) // 8 * 8
    tail_base = rows_per_tile * _NS
    tail_rows = n_nodes - tail_base
    per_tile = n_chunks * _CHUNK

    @functools.partial(
        pl.kernel,
        out_type=jax.ShapeDtypeStruct((n_nodes, 2 * d_half), jnp.float32),
        mesh=mesh,
        scratch_types=[
            pltpu.VMEM_SHARED((n_nodes, d_half), jnp.float32),
            pltpu.VMEM((per_tile,), jnp.int32),
            pltpu.VMEM((per_tile,), jnp.float32),
            pltpu.VMEM((_CHUNK,), jnp.int32),
            pltpu.VMEM((_CHUNK,), jnp.int32),
            pltpu.VMEM((_CHUNK,), jnp.int32),
            pltpu.VMEM((_CHUNK,), jnp.int32),
            pltpu.VMEM((_CHUNK, d_half), jnp.bfloat16),
            pltpu.VMEM((_CHUNK, d_half), jnp.bfloat16),
            pltpu.VMEM((_CHUNK, d_half), jnp.float32),
            pltpu.VMEM((_CHUNK, d_half), jnp.float32),
            pltpu.SemaphoreType.DMA,
            pltpu.SemaphoreType.DMA,
            pltpu.SemaphoreType.DMA,
            pltpu.SemaphoreType.DMA,
        ],
        compiler_params=pltpu.CompilerParams(use_tc_tiling_on_sc=False,
                                             needs_layout_passes=False),
    )
    def spmm(h_hbm, ep_hbm, w_hbm, zeros_hbm, out_hbm,
             acc_sh, ep_v, w_v, si0, si1, di0, di1, g0, g1, s0, s1,
             gsem0, gsem1, ssem0, ssem1):
        gbufs, sbufs = (g0, g1), (s0, s1)
        sidx, didx = (si0, si1), (di0, di1)
        gsems, ssems = (gsem0, gsem1), (ssem0, ssem1)
        c = lax.axis_index("c")
        s = lax.axis_index("s")
        row_ofs = c * n_nodes

        pltpu.sync_copy(zeros_hbm.at[pl.ds(s * rows_per_tile, rows_per_tile)],
                        acc_sh.at[pl.ds(s * rows_per_tile, rows_per_tile)])
        if tail_rows:
            @pl.when(s == _NS - 1)
            def _():
                pltpu.sync_copy(zeros_hbm.at[pl.ds(tail_base, tail_rows)],
                                acc_sh.at[pl.ds(tail_base, tail_rows)])

        pltpu.sync_copy(ep_hbm.at[pl.ds(s * per_tile, per_tile)], ep_v)
        pltpu.sync_copy(w_hbm.at[pl.ds(s * per_tile, per_tile)], w_v)
        plsc.subcore_barrier()

        def build_sidx(chunk, buf):
            for g16 in range(_CHUNK // 16):
                sl = pl.ds(g16 * 16, 16)
                v = ep_v[pl.ds(chunk * _CHUNK + g16 * 16, 16)]
                buf[sl] = (v & ((1 << _IDXBITS) - 1)) + row_ofs

        def build_didx(chunk, buf):
            for g16 in range(_CHUNK // 16):
                sl = pl.ds(g16 * 16, 16)
                v = ep_v[pl.ds(chunk * _CHUNK + g16 * 16, 16)]
                buf[sl] = lax.shift_right_logical(v, _IDXBITS)

        for b in range(2):
            build_sidx(jnp.int32(b), sidx[b])
            pltpu.async_copy(h_hbm.at[sidx[b]], gbufs[b], gsems[b])

        def pair_body(j2, carry):
            for b in range(2):
                chunk = j2 * 2 + b
                gb, sb = gbufs[b], sbufs[b]
                pltpu.make_async_copy(h_hbm.at[sidx[b]], gb, gsems[b]).wait()

                @pl.when(chunk >= 2)
                def _():
                    pltpu.make_async_copy(sb, acc_sh.at[didx[b]],
                                          ssems[b]).wait()

                for g in range(_CHUNK // 16):
                    wv = w_v[pl.ds(chunk * _CHUNK + g * 16, 16)]
                    for e16 in range(16):
                        e = g * 16 + e16
                        wb = _lane_bcast(wv, e16)
                        for f in range(d_half // 32):
                            v = plsc.bitcast(gb[e, pl.ds(f * 32, 32)],
                                             jnp.int32)
                            lo = plsc.bitcast(lax.shift_left(v, 16),
                                              jnp.float32)
                            hi = plsc.bitcast(v & jnp.int32(-65536),
                                              jnp.float32)
                            sb[e, pl.ds(f * 32, 16)] = lo * wb
                            sb[e, pl.ds(f * 32 + 16, 16)] = hi * wb

                @pl.when(chunk + 2 < n_chunks)
                def _():
                    build_sidx(chunk + 2, sidx[b])
                    pltpu.async_copy(h_hbm.at[sidx[b]], gb, gsems[b])

                build_didx(chunk, didx[b])
                pltpu.async_copy(sb, acc_sh.at[didx[b]], ssems[b], add=True)
            return carry

        lax.fori_loop(0, n_chunks // 2, pair_body, 0)
        for b in range(2):
            pltpu.make_async_copy(sbufs[b], acc_sh.at[didx[b]],
                                  ssems[b]).wait()

        plsc.subcore_barrier()
        col = pl.ds(c * d_half, d_half)
        pltpu.sync_copy(acc_sh.at[pl.ds(s * rows_per_tile, rows_per_tile)],
                        out_hbm.at[pl.ds(s * rows_per_tile, rows_per_tile),
                                   col])
        if tail_rows:
            @pl.when(s == _NS - 1)
            def _():
                pltpu.sync_copy(acc_sh.at[pl.ds(tail_base, tail_rows)],
                                out_hbm.at[pl.ds(tail_base, tail_rows), col])

    return spmm


def _pack_colorder(dh):
    co = np.empty(dh, np.int64)
    for f in range(dh // 32):
        for l in range(16):
            co[32 * f + 2 * l] = 32 * f + l
            co[32 * f + 2 * l + 1] = 32 * f + 16 + l
    return co


def _nbr_matmul(x, W_nbr, block_n=1000):
    n, d_in = x.shape
    d_out = W_nbr.shape[1]
    dh = d_out // 2

    co = _pack_colorder(dh)
    W2 = jnp.stack([W_nbr[:, :dh][:, co], W_nbr[:, dh:][:, co]])

    def body(x_ref, wn_ref, h_ref):
        h_ref[...] = jnp.dot(
            x_ref[...], wn_ref[0], preferred_element_type=jnp.float32
        ).astype(jnp.bfloat16)

    return pl.pallas_call(
        body,
        grid=(2, n // block_n),
        in_specs=[
            pl.BlockSpec((block_n, d_in), lambda j, i: (i, 0)),
            pl.BlockSpec((1, d_in, dh), lambda j, i: (j, 0, 0)),
        ],
        out_specs=pl.BlockSpec((block_n, dh),
                               lambda j, i: (j * (n // block_n) + i, 0)),
        out_shape=jax.ShapeDtypeStruct((2 * n, dh), jnp.bfloat16),
    )(x, W2)


def _edge_pack(edge_index, e_pad):
    _, e = edge_index.shape
    rows, rows_pad = e // 128, e_pad // 128

    def body(idx_ref, o_ref):
        o_ref[pl.ds(0, rows), :] = (
            idx_ref[1] | (idx_ref[0] << _IDXBITS))
        if rows_pad > rows:
            o_ref[pl.ds(rows, rows_pad - rows), :] = jnp.zeros(
                (rows_pad - rows, 128), jnp.int32)

    packed = pl.pallas_call(
        body,
        in_specs=[pl.BlockSpec((2, rows, 128), lambda: (0, 0, 0))],
        out_specs=pl.BlockSpec((rows_pad, 128), lambda: (0, 0)),
        out_shape=jax.ShapeDtypeStruct((rows_pad, 128), jnp.int32),
    )(edge_index.reshape(2, rows, 128))
    return packed.reshape(e_pad)


def _own_matmul(x, W_own, b, block_n=1000):
    n, d_in = x.shape
    d_out = W_own.shape[1]

    def body(x_ref, wo_ref, b_ref, o_ref):
        o_ref[...] = (
            jnp.dot(x_ref[...], wo_ref[...], preferred_element_type=jnp.float32)
            + b_ref[...]
        )

    return pl.pallas_call(
        body,
        grid=(n // block_n,),
        in_specs=[
            pl.BlockSpec((block_n, d_in), lambda i: (i, 0)),
            pl.BlockSpec((d_in, d_out), lambda i: (0, 0)),
            pl.BlockSpec((1, d_out), lambda i: (0, 0)),
        ],
        out_specs=pl.BlockSpec((block_n, d_out), lambda i: (i, 0)),
        out_shape=jax.ShapeDtypeStruct((n, d_out), jnp.float32),
    )(x, W_own, b.reshape(1, d_out))


def _mid_tanh(agg, own, block_n=1000):
    n, d = agg.shape

    def body(a_ref, own_ref, h_ref):
        h_ref[...] = jnp.tanh(a_ref[...] + own_ref[...])

    return pl.pallas_call(
        body,
        grid=(n // block_n,),
        in_specs=[
            pl.BlockSpec((block_n, d), lambda i: (i, 0)),
            pl.BlockSpec((block_n, d), lambda i: (i, 0)),
        ],
        out_specs=pl.BlockSpec((block_n, d), lambda i: (i, 0)),
        out_shape=jax.ShapeDtypeStruct((n, d), jnp.float32),
    )(agg, own)


def _final_add(agg, own, block_n=1000):
    n, d = agg.shape

    def body(a_ref, own_ref, o_ref):
        o_ref[...] = a_ref[...] + own_ref[...]

    return pl.pallas_call(
        body,
        grid=(n // block_n,),
        in_specs=[
            pl.BlockSpec((block_n, d), lambda i: (i, 0)),
            pl.BlockSpec((block_n, d), lambda i: (i, 0)),
        ],
        out_specs=pl.BlockSpec((block_n, d), lambda i: (i, 0)),
        out_shape=jax.ShapeDtypeStruct((n, d), jnp.float32),
    )(agg, own)


def kernel(x, edge_index, edge_weight, W_own0, W_nbr0, b0, W_own1, W_nbr1, b1):
    n, d_in = x.shape
    e = edge_weight.shape[0]
    d_hid = W_nbr0.shape[1]
    d_out = W_nbr1.shape[1]

    n_chunks = -(-e // (_NS * _CHUNK))
    n_chunks += n_chunks % 2
    per_tile = n_chunks * _CHUNK
    e_pad = per_tile * _NS

    pad = e_pad - e
    w = jnp.concatenate([edge_weight.astype(jnp.float32),
                         jnp.zeros((pad,), jnp.float32)])
    ep = _edge_pack(edge_index.astype(jnp.int32), e_pad)

    zeros_hid = jnp.zeros((n, d_hid // 2), jnp.float32)
    zeros_out = jnp.zeros((n, d_out // 2), jnp.float32)

    spmm0 = _make_spmm(n, d_hid // 2, n_chunks)
    spmm1 = _make_spmm(n, d_out // 2, n_chunks)

    h0 = _nbr_matmul(x, W_nbr0)
    agg0 = spmm0(h0, ep, w, zeros_hid)
    own0 = _own_matmul(x, W_own0, b0)
    h = _mid_tanh(agg0, own0)
    h1 = _nbr_matmul(h, W_nbr1)
    agg1 = spmm1(h1, ep, w, zeros_out)
    own1 = _own_matmul(h, W_own1, b1)
    return _final_add(agg1, own1)


# speedup vs baseline: 2.5619x; 1.0340x over previous
import functools

import jax
import jax.numpy as jnp
import numpy as np
from jax import lax
from jax.experimental import pallas as pl
from jax.experimental.pallas import tpu as pltpu
from jax.experimental.pallas import tpu_sc as plsc


def _lane_bcast(vec, lane):
    idx = jnp.full((16, 1), lane, jnp.int32)
    dnums = lax.GatherDimensionNumbers(
        offset_dims=(), collapsed_slice_dims=(0,), start_index_map=(0,))
    return lax.gather(vec, idx, dnums, (1,),
                      mode=lax.GatherScatterMode.PROMISE_IN_BOUNDS)


_NC = 2
_NS = 16
_CHUNK = 128
_IDXBITS = 14


def _make_spmm(n_nodes, d_half, n_chunks, d_out_pad=None):
    assert n_chunks % 2 == 0
    mesh = plsc.VectorSubcoreMesh(core_axis_name="c", subcore_axis_name="s",
                                  num_cores=_NC, num_subcores=_NS)
    rows_per_tile = (n_nodes // _NS) // 8 * 8
    tail_base = rows_per_tile * _NS
    tail_rows = n_nodes - tail_base
    per_tile = n_chunks * _CHUNK
    d_out_pad = d_out_pad or 2 * d_half

    @functools.partial(
        pl.kernel,
        out_type=jax.ShapeDtypeStruct((n_nodes, d_out_pad), jnp.float32),
        mesh=mesh,
        scratch_types=[
            pltpu.VMEM_SHARED((n_nodes, d_half), jnp.float32),
            pltpu.VMEM((per_tile,), jnp.int32),
            pltpu.VMEM((per_tile,), jnp.float32),
            pltpu.VMEM((_CHUNK,), jnp.int32),
            pltpu.VMEM((_CHUNK,), jnp.int32),
            pltpu.VMEM((_CHUNK,), jnp.int32),
            pltpu.VMEM((_CHUNK,), jnp.int32),
            pltpu.VMEM((_CHUNK, d_half), jnp.bfloat16),
            pltpu.VMEM((_CHUNK, d_half), jnp.bfloat16),
            pltpu.VMEM((_CHUNK, d_half), jnp.float32),
            pltpu.VMEM((_CHUNK, d_half), jnp.float32),
            pltpu.SemaphoreType.DMA,
            pltpu.SemaphoreType.DMA,
            pltpu.SemaphoreType.DMA,
            pltpu.SemaphoreType.DMA,
        ],
        compiler_params=pltpu.CompilerParams(use_tc_tiling_on_sc=False,
                                             needs_layout_passes=False),
    )
    def spmm(h_hbm, ep_hbm, w_hbm, zeros_hbm, out_hbm,
             acc_sh, ep_v, w_v, si0, si1, di0, di1, g0, g1, s0, s1,
             gsem0, gsem1, ssem0, ssem1):
        gbufs, sbufs = (g0, g1), (s0, s1)
        sidx, didx = (si0, si1), (di0, di1)
        gsems, ssems = (gsem0, gsem1), (ssem0, ssem1)
        c = lax.axis_index("c")
        s = lax.axis_index("s")
        row_ofs = c * n_nodes

        pltpu.sync_copy(zeros_hbm.at[pl.ds(s * rows_per_tile, rows_per_tile)],
                        acc_sh.at[pl.ds(s * rows_per_tile, rows_per_tile)])
        if tail_rows:
            @pl.when(s == _NS - 1)
            def _():
                pltpu.sync_copy(zeros_hbm.at[pl.ds(tail_base, tail_rows)],
                                acc_sh.at[pl.ds(tail_base, tail_rows)])

        pltpu.sync_copy(ep_hbm.at[pl.ds(s * per_tile, per_tile)], ep_v)
        pltpu.sync_copy(w_hbm.at[pl.ds(s * per_tile, per_tile)], w_v)
        plsc.subcore_barrier()

        def build_sidx(chunk, buf):
            for g16 in range(_CHUNK // 16):
                sl = pl.ds(g16 * 16, 16)
                v = ep_v[pl.ds(chunk * _CHUNK + g16 * 16, 16)]
                buf[sl] = (v & ((1 << _IDXBITS) - 1)) + row_ofs

        def build_didx(chunk, buf):
            for g16 in range(_CHUNK // 16):
                sl = pl.ds(g16 * 16, 16)
                v = ep_v[pl.ds(chunk * _CHUNK + g16 * 16, 16)]
                buf[sl] = lax.shift_right_logical(v, _IDXBITS)

        for b in range(2):
            build_sidx(jnp.int32(b), sidx[b])
            pltpu.async_copy(h_hbm.at[sidx[b]], gbufs[b], gsems[b])

        def pair_body(j2, carry):
            for b in range(2):
                chunk = j2 * 2 + b
                gb, sb = gbufs[b], sbufs[b]
                pltpu.make_async_copy(h_hbm.at[sidx[b]], gb, gsems[b]).wait()

                @pl.when(chunk >= 2)
                def _():
                    pltpu.make_async_copy(sb, acc_sh.at[didx[b]],
                                          ssems[b]).wait()

                for g in range(_CHUNK // 16):
                    wv = w_v[pl.ds(chunk * _CHUNK + g * 16, 16)]
                    for e16 in range(16):
                        e = g * 16 + e16
                        wb = _lane_bcast(wv, e16)
                        for f in range(d_half // 32):
                            v = plsc.bitcast(gb[e, pl.ds(f * 32, 32)],
                                             jnp.int32)
                            lo = plsc.bitcast(lax.shift_left(v, 16),
                                              jnp.float32)
                            hi = plsc.bitcast(v & jnp.int32(-65536),
                                              jnp.float32)
                            sb[e, pl.ds(f * 32, 16)] = lo * wb
                            sb[e, pl.ds(f * 32 + 16, 16)] = hi * wb

                @pl.when(chunk + 2 < n_chunks)
                def _():
                    build_sidx(chunk + 2, sidx[b])
                    pltpu.async_copy(h_hbm.at[sidx[b]], gb, gsems[b])

                build_didx(chunk, didx[b])
                pltpu.async_copy(sb, acc_sh.at[didx[b]], ssems[b], add=True)
            return carry

        lax.fori_loop(0, n_chunks // 2, pair_body, 0)
        for b in range(2):
            pltpu.make_async_copy(sbufs[b], acc_sh.at[didx[b]],
                                  ssems[b]).wait()

        plsc.subcore_barrier()
        col = pl.ds(c * d_half, d_half)
        pltpu.sync_copy(acc_sh.at[pl.ds(s * rows_per_tile, rows_per_tile)],
                        out_hbm.at[pl.ds(s * rows_per_tile, rows_per_tile),
                                   col])
        if tail_rows:
            @pl.when(s == _NS - 1)
            def _():
                pltpu.sync_copy(acc_sh.at[pl.ds(tail_base, tail_rows)],
                                out_hbm.at[pl.ds(tail_base, tail_rows), col])

    return spmm


def _pack_colorder(dh):
    co = np.empty(dh, np.int64)
    for f in range(dh // 32):
        for l in range(16):
            co[32 * f + 2 * l] = 32 * f + l
            co[32 * f + 2 * l + 1] = 32 * f + 16 + l
    return co


def _nbr_matmul(x, W_nbr, block_n=1000, tanh_in=False):
    xs = x if isinstance(x, tuple) else (x,)
    n, d_in = xs[0].shape
    d_out = W_nbr.shape[1]
    dh = d_out // 2

    co = _pack_colorder(dh)
    W2 = jnp.stack([W_nbr[:, :dh][:, co], W_nbr[:, dh:][:, co]])

    def body(*refs):
        (*x_refs, wn_ref, h_ref) = refs
        if tanh_in:
            xb = jnp.tanh(x_refs[0][...] + x_refs[1][...])
        else:
            xb = x_refs[0][...]
        h_ref[...] = jnp.dot(
            xb, wn_ref[0], preferred_element_type=jnp.float32
        ).astype(jnp.bfloat16)

    return pl.pallas_call(
        body,
        grid=(2, n // block_n),
        in_specs=[
            *[pl.BlockSpec((block_n, d_in), lambda j, i: (i, 0)) for _ in xs],
            pl.BlockSpec((1, d_in, dh), lambda j, i: (j, 0, 0)),
        ],
        out_specs=pl.BlockSpec((block_n, dh),
                               lambda j, i: (j * (n // block_n) + i, 0)),
        out_shape=jax.ShapeDtypeStruct((2 * n, dh), jnp.bfloat16),
    )(*xs, W2)


def _edge_pack(edge_index, edge_weight, e_pad):
    _, e = edge_index.shape
    rows, rows_pad = e // 128, e_pad // 128

    def body(idx_ref, w_ref, o_ref, wo_ref):
        o_ref[pl.ds(0, rows), :] = idx_ref[1] | (idx_ref[0] << _IDXBITS)
        wo_ref[pl.ds(0, rows), :] = w_ref[...]
        if rows_pad > rows:
            zrows = rows_pad - rows
            o_ref[pl.ds(rows, zrows), :] = jnp.zeros((zrows, 128), jnp.int32)
            wo_ref[pl.ds(rows, zrows), :] = jnp.zeros((zrows, 128),
                                                      jnp.float32)

    packed, wp = pl.pallas_call(
        body,
        in_specs=[
            pl.BlockSpec((2, rows, 128), lambda: (0, 0, 0)),
            pl.BlockSpec((rows, 128), lambda: (0, 0)),
        ],
        out_specs=[
            pl.BlockSpec((rows_pad, 128), lambda: (0, 0)),
            pl.BlockSpec((rows_pad, 128), lambda: (0, 0)),
        ],
        out_shape=[
            jax.ShapeDtypeStruct((rows_pad, 128), jnp.int32),
            jax.ShapeDtypeStruct((rows_pad, 128), jnp.float32),
        ],
    )(edge_index.reshape(2, rows, 128),
      edge_weight.astype(jnp.float32).reshape(rows, 128))
    return packed.reshape(e_pad), wp.reshape(e_pad)


def _own_matmul(x, W_own, b, block_n=1000, tanh_in=False):
    xs = x if isinstance(x, tuple) else (x,)
    n, d_in = xs[0].shape
    d_out = W_own.shape[1]

    def body(*refs):
        (*x_refs, wo_ref, b_ref, o_ref) = refs
        if tanh_in:
            xb = jnp.tanh(x_refs[0][...] + x_refs[1][...])
        else:
            xb = x_refs[0][...]
        o_ref[...] = (
            jnp.dot(xb, wo_ref[...], preferred_element_type=jnp.float32)
            + b_ref[...]
        )

    return pl.pallas_call(
        body,
        grid=(n // block_n,),
        in_specs=[
            *[pl.BlockSpec((block_n, d_in), lambda i: (i, 0)) for _ in xs],
            pl.BlockSpec((d_in, d_out), lambda i: (0, 0)),
            pl.BlockSpec((1, d_out), lambda i: (0, 0)),
        ],
        out_specs=pl.BlockSpec((block_n, d_out), lambda i: (i, 0)),
        out_shape=jax.ShapeDtypeStruct((n, d_out), jnp.float32),
    )(*xs, W_own, b.reshape(1, d_out))


def _final_add(agg, own, block_n=1000):
    n, d = own.shape
    dp = agg.shape[1]

    def body(a_ref, own_ref, o_ref):
        o_ref[...] = a_ref[:, :d] + own_ref[...]

    return pl.pallas_call(
        body,
        grid=(n // block_n,),
        in_specs=[
            pl.BlockSpec((block_n, dp), lambda i: (i, 0)),
            pl.BlockSpec((block_n, d), lambda i: (i, 0)),
        ],
        out_specs=pl.BlockSpec((block_n, d), lambda i: (i, 0)),
        out_shape=jax.ShapeDtypeStruct((n, d), jnp.float32),
    )(agg, own)


def kernel(x, edge_index, edge_weight, W_own0, W_nbr0, b0, W_own1, W_nbr1, b1):
    n, d_in = x.shape
    e = edge_weight.shape[0]
    d_hid = W_nbr0.shape[1]
    d_out = W_nbr1.shape[1]

    n_chunks = -(-e // (_NS * _CHUNK))
    n_chunks += n_chunks % 2
    per_tile = n_chunks * _CHUNK
    e_pad = per_tile * _NS

    ep, w = _edge_pack(edge_index.astype(jnp.int32), edge_weight, e_pad)

    zeros_hid = jnp.zeros((n, d_hid // 2), jnp.float32)
    zeros_out = jnp.zeros((n, d_out // 2), jnp.float32)

    spmm0 = _make_spmm(n, d_hid // 2, n_chunks)
    spmm1 = _make_spmm(n, d_out // 2, n_chunks, d_out_pad=128)

    h0 = _nbr_matmul(x, W_nbr0)
    agg0 = spmm0(h0, ep, w, zeros_hid)
    own0 = _own_matmul(x, W_own0, b0)
    h1 = _nbr_matmul((agg0, own0), W_nbr1, tanh_in=True)
    agg1 = spmm1(h1, ep, w, zeros_out)
    own1 = _own_matmul((agg0, own0), W_own1, b1, tanh_in=True)
    return _final_add(agg1, own1)
